# Initial kernel scaffold; baseline (speedup 1.0000x reference)
import functools
import math

import jax
import jax.numpy as jnp
from jax import lax
from jax.experimental import pallas as pl
from jax.experimental.pallas import tpu as pltpu
from jax.experimental.pallas import tpu_sc as plsc

N_NODES = 10000
N_EDGES = 320000
D = 128
D_EDGE = 16

_SC_INFO = plsc.get_sparse_core_info()
NC = _SC_INFO.num_cores
NS = _SC_INFO.num_subcores
NW = NC * NS

E_PER_TILE = N_EDGES // NW
CHUNK = 80
N_CHUNKS = E_PER_TILE // CHUNK
ROWS_PER_TILE = N_NODES // NS
ZROWS = 125


def _emb_body(a_ref, w_ref, b_ref, o_ref):
    o_ref[...] = (
        jnp.dot(a_ref[...], w_ref[...], preferred_element_type=jnp.float32)
        + b_ref[...]
    )


def _edge_embed(edge_attr, W_edge, b_edge):
    BE = 2000
    return pl.pallas_call(
        _emb_body,
        grid=(N_EDGES // BE,),
        in_specs=[
            pl.BlockSpec((BE, D_EDGE), lambda i: (i, 0)),
            pl.BlockSpec((D_EDGE, D), lambda i: (0, 0)),
            pl.BlockSpec((1, D), lambda i: (0, 0)),
        ],
        out_specs=pl.BlockSpec((BE, D), lambda i: (i, 0)),
        out_shape=jax.ShapeDtypeStruct((N_EDGES, D), jnp.float32),
    )(edge_attr, W_edge, b_edge.reshape(1, D))


_MESH = plsc.VectorSubcoreMesh(core_axis_name="c", subcore_axis_name="s")


@functools.partial(
    pl.kernel,
    mesh=_MESH,
    out_type=jax.ShapeDtypeStruct((NC, N_NODES, D), jnp.float32),
    scratch_types=[
        pltpu.VMEM((CHUNK,), jnp.int32),
        pltpu.VMEM((CHUNK,), jnp.int32),
        pltpu.VMEM((CHUNK, D), jnp.float32),
        pltpu.VMEM((CHUNK, D), jnp.float32),
        pltpu.VMEM((ZROWS, D), jnp.float32),
        pltpu.VMEM_SHARED((N_NODES, D), jnp.float32),
        pltpu.SemaphoreType.DMA,
    ],
)
def _sc_aggregate(node_hbm, emb_hbm, src_hbm, dst_hbm, out_hbm,
                  src_v, dst_v, rows_v, emb_v, zero_v, agg_sh, sem):
    c = lax.axis_index("c")
    s = lax.axis_index("s")

    def _zrow(i, carry):
        for j in range(D // 16):
            zero_v[i, pl.ds(j * 16, 16)] = jnp.zeros((16,), jnp.float32)
        return carry

    lax.fori_loop(0, ZROWS, _zrow, 0)

    def _zcopy(k, carry):
        pltpu.sync_copy(
            zero_v, agg_sh.at[pl.ds(s * ROWS_PER_TILE + k * ZROWS, ZROWS)]
        )
        return carry

    lax.fori_loop(0, ROWS_PER_TILE // ZROWS, _zcopy, 0)
    plsc.subcore_barrier()

    base = (c * NS + s) * E_PER_TILE

    def _chunk(i, carry):
        off = base + i * CHUNK
        pltpu.sync_copy(src_hbm.at[pl.ds(off, CHUNK)], src_v)
        pltpu.sync_copy(dst_hbm.at[pl.ds(off, CHUNK)], dst_v)
        gat = pltpu.async_copy(node_hbm.at[src_v], rows_v, sem)
        pltpu.sync_copy(emb_hbm.at[pl.ds(off, CHUNK)], emb_v)
        gat.wait()

        def _row(r, rc):
            for j in range(D // 16):
                sl = pl.ds(j * 16, 16)
                rows_v[r, sl] = jnp.maximum(rows_v[r, sl] + emb_v[r, sl], 0.0)
            return rc

        lax.fori_loop(0, CHUNK, _row, 0)
        pltpu.sync_copy(rows_v, agg_sh.at[dst_v], add=True)
        return carry

    lax.fori_loop(0, N_CHUNKS, _chunk, 0)
    plsc.subcore_barrier()

    pltpu.sync_copy(
        agg_sh.at[pl.ds(s * ROWS_PER_TILE, ROWS_PER_TILE)],
        out_hbm.at[c, pl.ds(s * ROWS_PER_TILE, ROWS_PER_TILE)],
    )


_INV_SQRT_N = 1.0 / math.sqrt(float(N_NODES))


def _out_body(agg_ref, x_ref, w_ref, b_ref, g_ref, bt_ref, o_ref):
    agg = agg_ref[0] + agg_ref[1]
    out = jnp.dot(agg, w_ref[...], preferred_element_type=jnp.float32) + b_ref[...]
    mean = jnp.mean(out, axis=-1, keepdims=True)
    var = jnp.mean((out - mean) ** 2, axis=-1, keepdims=True)
    out = (out - mean) * lax.rsqrt(var + 1e-5) * g_ref[...] + bt_ref[...]
    o_ref[...] = out * _INV_SQRT_N + x_ref[...]


def _finish(agg_parts, node_feat, W_lin, b_lin, gamma, beta):
    BN = 1000
    return pl.pallas_call(
        _out_body,
        grid=(N_NODES // BN,),
        in_specs=[
            pl.BlockSpec((NC, BN, D), lambda i: (0, i, 0)),
            pl.BlockSpec((BN, D), lambda i: (i, 0)),
            pl.BlockSpec((D, D), lambda i: (0, 0)),
            pl.BlockSpec((1, D), lambda i: (0, 0)),
            pl.BlockSpec((1, D), lambda i: (0, 0)),
            pl.BlockSpec((1, D), lambda i: (0, 0)),
        ],
        out_specs=pl.BlockSpec((BN, D), lambda i: (i, 0)),
        out_shape=jax.ShapeDtypeStruct((N_NODES, D), jnp.float32),
    )(agg_parts, node_feat, W_lin, b_lin.reshape(1, D),
      gamma.reshape(1, D), beta.reshape(1, D))


def kernel(node_feat, edge_attr, W_edge, b_edge, W_lin, b_lin, gamma, beta,
           edge_index):
    src = edge_index[0].astype(jnp.int32)
    dst = edge_index[1].astype(jnp.int32)
    emb = _edge_embed(edge_attr, W_edge, b_edge)
    agg_parts = _sc_aggregate(node_feat, emb, src, dst)
    return _finish(agg_parts, node_feat, W_lin, b_lin, gamma, beta)


# speedup vs baseline: 2.6045x; 2.6045x over previous
import functools
import math

import jax
import jax.numpy as jnp
from jax import lax
from jax.experimental import pallas as pl
from jax.experimental.pallas import tpu as pltpu
from jax.experimental.pallas import tpu_sc as plsc

N_NODES = 10000
N_EDGES = 320000
D = 128
D_EDGE = 16

_SC_INFO = plsc.get_sparse_core_info()
NC = _SC_INFO.num_cores
NS = _SC_INFO.num_subcores
NW = NC * NS

E_PER_TILE = N_EDGES // NW
CHUNK = 80
N_CHUNKS = E_PER_TILE // CHUNK
N_PAD = 10240
ROWS_PER_TILE = N_PAD // NS
ZROWS = 128


def _emb_body(a_ref, w_ref, b_ref, o_ref):
    o_ref[...] = (
        jnp.dot(a_ref[...], w_ref[...], preferred_element_type=jnp.float32)
        + b_ref[...]
    )


def _edge_embed(edge_attr, W_edge, b_edge):
    BE = 2000
    return pl.pallas_call(
        _emb_body,
        grid=(N_EDGES // BE,),
        in_specs=[
            pl.BlockSpec((BE, D_EDGE), lambda i: (i, 0)),
            pl.BlockSpec((D_EDGE, D), lambda i: (0, 0)),
            pl.BlockSpec((1, D), lambda i: (0, 0)),
        ],
        out_specs=pl.BlockSpec((BE, D), lambda i: (i, 0)),
        out_shape=jax.ShapeDtypeStruct((N_EDGES, D), jnp.float32),
    )(edge_attr, W_edge, b_edge.reshape(1, D))


_MESH = plsc.VectorSubcoreMesh(core_axis_name="c", subcore_axis_name="s")


@functools.partial(
    pl.kernel,
    mesh=_MESH,
    out_type=jax.ShapeDtypeStruct((NC, N_PAD, D), jnp.float32),
    scratch_types=[
        pltpu.VMEM((CHUNK,), jnp.int32),
        pltpu.VMEM((CHUNK,), jnp.int32),
        pltpu.VMEM((CHUNK, D), jnp.float32),
        pltpu.VMEM((CHUNK, D), jnp.float32),
        pltpu.VMEM((ZROWS, D), jnp.float32),
        pltpu.VMEM_SHARED((N_PAD, D), jnp.float32),
        pltpu.SemaphoreType.DMA,
    ],
)
def _sc_aggregate(node_hbm, emb_hbm, src_hbm, dst_hbm, out_hbm,
                  src_v, dst_v, rows_v, emb_v, zero_v, agg_sh, sem):
    c = lax.axis_index("c")
    s = lax.axis_index("s")

    def _zrow(i, carry):
        for j in range(D // 16):
            zero_v[i, pl.ds(j * 16, 16)] = jnp.zeros((16,), jnp.float32)
        return carry

    lax.fori_loop(0, ZROWS, _zrow, 0)

    def _zcopy(k, carry):
        pltpu.sync_copy(
            zero_v, agg_sh.at[pl.ds(s * ROWS_PER_TILE + k * ZROWS, ZROWS)]
        )
        return carry

    lax.fori_loop(0, ROWS_PER_TILE // ZROWS, _zcopy, 0)
    plsc.subcore_barrier()

    base = (c * NS + s) * E_PER_TILE

    def _chunk(i, carry):
        off = base + i * CHUNK
        pltpu.sync_copy(src_hbm.at[pl.ds(off, CHUNK)], src_v)
        pltpu.sync_copy(dst_hbm.at[pl.ds(off, CHUNK)], dst_v)
        gat = pltpu.async_copy(node_hbm.at[src_v], rows_v, sem)
        pltpu.sync_copy(emb_hbm.at[pl.ds(off, CHUNK)], emb_v)
        gat.wait()

        def _row(r, rc):
            for j in range(D // 16):
                sl = pl.ds(j * 16, 16)
                rows_v[r, sl] = jnp.maximum(rows_v[r, sl] + emb_v[r, sl], 0.0)
            return rc

        lax.fori_loop(0, CHUNK, _row, 0)
        pltpu.sync_copy(rows_v, agg_sh.at[dst_v], add=True)
        return carry

    lax.fori_loop(0, N_CHUNKS, _chunk, 0)
    plsc.subcore_barrier()

    pltpu.sync_copy(
        agg_sh.at[pl.ds(s * ROWS_PER_TILE, ROWS_PER_TILE)],
        out_hbm.at[c, pl.ds(s * ROWS_PER_TILE, ROWS_PER_TILE)],
    )


_INV_SQRT_N = 1.0 / math.sqrt(float(N_NODES))


def _out_body(agg_ref, x_ref, w_ref, b_ref, g_ref, bt_ref, o_ref):
    agg = agg_ref[0] + agg_ref[1]
    out = jnp.dot(agg, w_ref[...], preferred_element_type=jnp.float32) + b_ref[...]
    mean = jnp.mean(out, axis=-1, keepdims=True)
    var = jnp.mean((out - mean) ** 2, axis=-1, keepdims=True)
    out = (out - mean) * lax.rsqrt(var + 1e-5) * g_ref[...] + bt_ref[...]
    o_ref[...] = out * _INV_SQRT_N + x_ref[...]


def _finish(agg_parts, node_feat, W_lin, b_lin, gamma, beta):
    BN = 1000
    return pl.pallas_call(
        _out_body,
        grid=(N_NODES // BN,),
        in_specs=[
            pl.BlockSpec((NC, BN, D), lambda i: (0, i, 0)),
            pl.BlockSpec((BN, D), lambda i: (i, 0)),
            pl.BlockSpec((D, D), lambda i: (0, 0)),
            pl.BlockSpec((1, D), lambda i: (0, 0)),
            pl.BlockSpec((1, D), lambda i: (0, 0)),
            pl.BlockSpec((1, D), lambda i: (0, 0)),
        ],
        out_specs=pl.BlockSpec((BN, D), lambda i: (i, 0)),
        out_shape=jax.ShapeDtypeStruct((N_NODES, D), jnp.float32),
    )(agg_parts, node_feat, W_lin, b_lin.reshape(1, D),
      gamma.reshape(1, D), beta.reshape(1, D))


def kernel(node_feat, edge_attr, W_edge, b_edge, W_lin, b_lin, gamma, beta,
           edge_index):
    src = edge_index[0].astype(jnp.int32)
    dst = edge_index[1].astype(jnp.int32)
    emb = _edge_embed(edge_attr, W_edge, b_edge)
    agg_parts = _sc_aggregate(node_feat, emb, src, dst)
    return _finish(agg_parts, node_feat, W_lin, b_lin, gamma, beta)


# speedup vs baseline: 3.9060x; 1.4997x over previous
import functools
import math

import jax
import jax.numpy as jnp
from jax import lax
from jax.experimental import pallas as pl
from jax.experimental.pallas import tpu as pltpu
from jax.experimental.pallas import tpu_sc as plsc

N_NODES = 10000
N_EDGES = 320000
D = 128
D_EDGE = 16

_SC_INFO = plsc.get_sparse_core_info()
NC = _SC_INFO.num_cores
NS = _SC_INFO.num_subcores
NW = NC * NS

E_PER_TILE = N_EDGES // NW
CHUNK = 40
N_CHUNKS = E_PER_TILE // CHUNK
N_PAD = 10240
ROWS_PER_TILE = N_PAD // NS

NSRC = 4
NDST = 8


def _emb_body(a_ref, w_ref, b_ref, o_ref):
    o_ref[...] = (
        jnp.dot(a_ref[...], w_ref[...], preferred_element_type=jnp.float32)
        + b_ref[...]
    )


def _edge_embed(edge_attr, W_edge, b_edge):
    BE = 2000
    return pl.pallas_call(
        _emb_body,
        grid=(N_EDGES // BE,),
        in_specs=[
            pl.BlockSpec((BE, D_EDGE), lambda i: (i, 0)),
            pl.BlockSpec((D_EDGE, D), lambda i: (0, 0)),
            pl.BlockSpec((1, D), lambda i: (0, 0)),
        ],
        out_specs=pl.BlockSpec((BE, D), lambda i: (i, 0)),
        out_shape=jax.ShapeDtypeStruct((N_EDGES, D), jnp.float32),
    )(edge_attr, W_edge, b_edge.reshape(1, D))


_MESH = plsc.VectorSubcoreMesh(core_axis_name="c", subcore_axis_name="s")

_SCRATCH = (
    [pltpu.VMEM((CHUNK, D), jnp.float32) for _ in range(6)]
    + [pltpu.VMEM((CHUNK,), jnp.int32) for _ in range(NSRC)]
    + [pltpu.VMEM((CHUNK,), jnp.int32) for _ in range(NDST)]
    + [pltpu.VMEM_SHARED((N_PAD, D), jnp.float32)]
    + [pltpu.SemaphoreType.DMA for _ in range(6 + NSRC + NDST)]
)


@functools.partial(
    pl.kernel,
    mesh=_MESH,
    out_type=jax.ShapeDtypeStruct((NC, N_PAD, D), jnp.float32),
    scratch_types=_SCRATCH,
)
def _sc_aggregate(node_hbm, emb_hbm, src_hbm, dst_hbm, out_hbm, *refs):
    rows = refs[0:2]
    embv = refs[2:4]
    msgv = refs[4:6]
    srcb = refs[6:6 + NSRC]
    dstb = refs[6 + NSRC:6 + NSRC + NDST]
    agg_sh = refs[6 + NSRC + NDST]
    sems = refs[7 + NSRC + NDST:]
    gsem = sems[0:2]
    esem = sems[2:4]
    ssem = sems[4:6]
    sis = sems[6:6 + NSRC]
    dis = sems[6 + NSRC:6 + NSRC + NDST]

    c = lax.axis_index("c")
    s = lax.axis_index("s")
    w = c * NS + s
    base = w * E_PER_TILE

    def issue_idx(i, stat):
        pltpu.async_copy(
            src_hbm.at[pl.ds(base + i * CHUNK, CHUNK)], srcb[stat % NSRC],
            sis[stat % NSRC],
        )
        pltpu.async_copy(
            dst_hbm.at[pl.ds(base + i * CHUNK, CHUNK)], dstb[stat % NDST],
            dis[stat % NDST],
        )

    def wait_src(stat):
        pltpu.make_async_copy(
            src_hbm.at[pl.ds(0, CHUNK)], srcb[stat % NSRC], sis[stat % NSRC]
        ).wait()

    def wait_dst(stat):
        pltpu.make_async_copy(
            dst_hbm.at[pl.ds(0, CHUNK)], dstb[stat % NDST], dis[stat % NDST]
        ).wait()

    def issue_data(i, stat):
        b = stat % 2
        pltpu.async_copy(node_hbm.at[srcb[stat % NSRC]], rows[b], gsem[b])
        pltpu.async_copy(
            emb_hbm.at[pl.ds(base + i * CHUNK, CHUNK)], embv[b], esem[b]
        )

    def wait_data(i, stat):
        b = stat % 2
        pltpu.make_async_copy(
            node_hbm.at[srcb[stat % NSRC]], rows[b], gsem[b]
        ).wait()
        pltpu.make_async_copy(
            emb_hbm.at[pl.ds(base + i * CHUNK, CHUNK)], embv[b], esem[b]
        ).wait()

    def issue_scatter(stat):
        b = stat % 2
        pltpu.async_copy(
            msgv[b], agg_sh.at[dstb[stat % NDST]], ssem[b], add=True
        )

    def wait_scatter(stat):
        b = stat % 2
        pltpu.make_async_copy(
            msgv[b], agg_sh.at[dstb[stat % NDST]], ssem[b]
        ).wait()

    def compute(stat):
        b = stat % 2
        r_buf, e_buf, m_buf = rows[b], embv[b], msgv[b]

        def _row(r, rc):
            for rr in range(2):
                row = r * 2 + rr
                for j in range(D // 16):
                    sl = pl.ds(j * 16, 16)
                    m_buf[row, sl] = jnp.maximum(
                        r_buf[row, sl] + e_buf[row, sl], 0.0
                    )
            return rc

        lax.fori_loop(0, CHUNK // 2, _row, 0)

    issue_idx(0, 0)
    issue_idx(1, 1)
    issue_idx(2, 2)

    def _zrow(i, carry):
        for j in range(D // 16):
            msgv[0][i, pl.ds(j * 16, 16)] = jnp.zeros((16,), jnp.float32)
        return carry

    lax.fori_loop(0, CHUNK, _zrow, 0)

    def _zcopy(k, carry):
        pltpu.sync_copy(
            msgv[0], agg_sh.at[pl.ds(s * ROWS_PER_TILE + k * CHUNK, CHUNK)]
        )
        return carry

    lax.fori_loop(0, ROWS_PER_TILE // CHUNK, _zcopy, 0)

    wait_src(0)
    issue_data(0, 0)
    wait_src(1)
    issue_data(1, 1)
    plsc.subcore_barrier()

    wait_data(0, 0)
    compute(0)
    wait_dst(0)
    issue_scatter(0)
    wait_src(2)
    issue_data(2, 2)
    issue_idx(3, 3)

    wait_data(1, 1)
    compute(1)
    wait_dst(1)
    issue_scatter(1)
    wait_src(3)
    issue_data(3, 3)
    issue_idx(4, 4)

    def _block(i, stat):
        wait_data(i, stat)
        wait_scatter(stat)
        compute(stat)
        wait_dst(stat)
        issue_scatter(stat)

        @pl.when(i + 2 < N_CHUNKS)
        def _():
            wait_src(stat + 2)
            issue_data(i + 2, stat + 2)

        @pl.when(i + 3 < N_CHUNKS)
        def _():
            issue_idx(i + 3, stat + 3)

    def _oct(g, carry):
        i0 = 2 + 8 * g
        for k in range(8):
            _block(i0 + k, 2 + k)
        return carry

    lax.fori_loop(0, (N_CHUNKS - 2) // 8, _oct, 0)

    wait_scatter(0)
    wait_scatter(1)
    plsc.subcore_barrier()

    pltpu.sync_copy(
        agg_sh.at[pl.ds(s * ROWS_PER_TILE, ROWS_PER_TILE)],
        out_hbm.at[c, pl.ds(s * ROWS_PER_TILE, ROWS_PER_TILE)],
    )


_INV_SQRT_N = 1.0 / math.sqrt(float(N_NODES))


def _out_body(agg_ref, x_ref, w_ref, b_ref, g_ref, bt_ref, o_ref):
    agg = agg_ref[0] + agg_ref[1]
    out = jnp.dot(agg, w_ref[...], preferred_element_type=jnp.float32) + b_ref[...]
    mean = jnp.mean(out, axis=-1, keepdims=True)
    var = jnp.mean((out - mean) ** 2, axis=-1, keepdims=True)
    out = (out - mean) * lax.rsqrt(var + 1e-5) * g_ref[...] + bt_ref[...]
    o_ref[...] = out * _INV_SQRT_N + x_ref[...]


def _finish(agg_parts, node_feat, W_lin, b_lin, gamma, beta):
    BN = 1000
    return pl.pallas_call(
        _out_body,
        grid=(N_NODES // BN,),
        in_specs=[
            pl.BlockSpec((NC, BN, D), lambda i: (0, i, 0)),
            pl.BlockSpec((BN, D), lambda i: (i, 0)),
            pl.BlockSpec((D, D), lambda i: (0, 0)),
            pl.BlockSpec((1, D), lambda i: (0, 0)),
            pl.BlockSpec((1, D), lambda i: (0, 0)),
            pl.BlockSpec((1, D), lambda i: (0, 0)),
        ],
        out_specs=pl.BlockSpec((BN, D), lambda i: (i, 0)),
        out_shape=jax.ShapeDtypeStruct((N_NODES, D), jnp.float32),
    )(agg_parts, node_feat, W_lin, b_lin.reshape(1, D),
      gamma.reshape(1, D), beta.reshape(1, D))


def kernel(node_feat, edge_attr, W_edge, b_edge, W_lin, b_lin, gamma, beta,
           edge_index):
    src = edge_index[0].astype(jnp.int32)
    dst = edge_index[1].astype(jnp.int32)
    emb = _edge_embed(edge_attr, W_edge, b_edge)
    agg_parts = _sc_aggregate(node_feat, emb, src, dst)
    return _finish(agg_parts, node_feat, W_lin, b_lin, gamma, beta)


# speedup vs baseline: 4.5135x; 1.1555x over previous
import functools
import math

import jax
import jax.numpy as jnp
from jax import lax
from jax.experimental import pallas as pl
from jax.experimental.pallas import tpu as pltpu
from jax.experimental.pallas import tpu_sc as plsc

N_NODES = 10000
N_EDGES = 320000
D = 128
D_EDGE = 16

_SC_INFO = plsc.get_sparse_core_info()
NC = _SC_INFO.num_cores
NS = _SC_INFO.num_subcores
NW = NC * NS

E_PER_TILE = N_EDGES // NW
CHUNK = 40
N_CHUNKS = E_PER_TILE // CHUNK
N_PAD = 10240
ROWS_PER_TILE = N_PAD // NS

NSRC = 4
NDST = 8


def _emb_body(a_ref, w_ref, b_ref, o_ref):
    o_ref[...] = (
        jnp.dot(a_ref[...], w_ref[...], preferred_element_type=jnp.float32)
        + b_ref[...]
    )


def _edge_embed(edge_attr, W_edge, b_edge):
    BE = 16000
    return pl.pallas_call(
        _emb_body,
        grid=(N_EDGES // BE,),
        in_specs=[
            pl.BlockSpec((BE, D_EDGE), lambda i: (i, 0)),
            pl.BlockSpec((D_EDGE, D), lambda i: (0, 0)),
            pl.BlockSpec((1, D), lambda i: (0, 0)),
        ],
        out_specs=pl.BlockSpec((BE, D), lambda i: (i, 0)),
        out_shape=jax.ShapeDtypeStruct((N_EDGES, D), jnp.float32),
    )(edge_attr, W_edge, b_edge.reshape(1, D))


_MESH = plsc.VectorSubcoreMesh(core_axis_name="c", subcore_axis_name="s")

_SCRATCH = (
    [pltpu.VMEM((CHUNK, D), jnp.float32) for _ in range(6)]
    + [pltpu.VMEM((CHUNK,), jnp.int32) for _ in range(NSRC)]
    + [pltpu.VMEM((CHUNK,), jnp.int32) for _ in range(NDST)]
    + [pltpu.VMEM_SHARED((N_PAD, D), jnp.float32)]
    + [pltpu.SemaphoreType.DMA for _ in range(6 + NSRC + NDST)]
)


@functools.partial(
    pl.kernel,
    mesh=_MESH,
    out_type=jax.ShapeDtypeStruct((NC, N_PAD, D), jnp.float32),
    scratch_types=_SCRATCH,
)
def _sc_aggregate(node_hbm, emb_hbm, src_hbm, dst_hbm, out_hbm, *refs):
    rows = refs[0:2]
    embv = refs[2:4]
    msgv = refs[4:6]
    srcb = refs[6:6 + NSRC]
    dstb = refs[6 + NSRC:6 + NSRC + NDST]
    agg_sh = refs[6 + NSRC + NDST]
    sems = refs[7 + NSRC + NDST:]
    gsem = sems[0:2]
    esem = sems[2:4]
    ssem = sems[4:6]
    sis = sems[6:6 + NSRC]
    dis = sems[6 + NSRC:6 + NSRC + NDST]

    c = lax.axis_index("c")
    s = lax.axis_index("s")
    w = c * NS + s
    base = w * E_PER_TILE

    def issue_idx(i, stat):
        pltpu.async_copy(
            src_hbm.at[pl.ds(base + i * CHUNK, CHUNK)], srcb[stat % NSRC],
            sis[stat % NSRC],
        )
        pltpu.async_copy(
            dst_hbm.at[pl.ds(base + i * CHUNK, CHUNK)], dstb[stat % NDST],
            dis[stat % NDST],
        )

    def wait_src(stat):
        pltpu.make_async_copy(
            src_hbm.at[pl.ds(0, CHUNK)], srcb[stat % NSRC], sis[stat % NSRC]
        ).wait()

    def wait_dst(stat):
        pltpu.make_async_copy(
            dst_hbm.at[pl.ds(0, CHUNK)], dstb[stat % NDST], dis[stat % NDST]
        ).wait()

    def issue_data(i, stat):
        b = stat % 2
        pltpu.async_copy(node_hbm.at[srcb[stat % NSRC]], rows[b], gsem[b])
        pltpu.async_copy(
            emb_hbm.at[pl.ds(base + i * CHUNK, CHUNK)], embv[b], esem[b]
        )

    def wait_data(i, stat):
        b = stat % 2
        pltpu.make_async_copy(
            node_hbm.at[srcb[stat % NSRC]], rows[b], gsem[b]
        ).wait()
        pltpu.make_async_copy(
            emb_hbm.at[pl.ds(base + i * CHUNK, CHUNK)], embv[b], esem[b]
        ).wait()

    def issue_scatter(stat):
        b = stat % 2
        pltpu.async_copy(
            msgv[b], agg_sh.at[dstb[stat % NDST]], ssem[b], add=True
        )

    def wait_scatter(stat):
        b = stat % 2
        pltpu.make_async_copy(
            msgv[b], agg_sh.at[dstb[stat % NDST]], ssem[b]
        ).wait()

    def compute(stat):
        b = stat % 2
        r_buf, e_buf, m_buf = rows[b], embv[b], msgv[b]

        def _row(r, rc):
            for rr in range(2):
                row = r * 2 + rr
                for j in range(D // 16):
                    sl = pl.ds(j * 16, 16)
                    m_buf[row, sl] = jnp.maximum(
                        r_buf[row, sl] + e_buf[row, sl], 0.0
                    )
            return rc

        lax.fori_loop(0, CHUNK // 2, _row, 0)

    issue_idx(0, 0)
    issue_idx(1, 1)
    issue_idx(2, 2)

    def _zrow(i, carry):
        for j in range(D // 16):
            msgv[0][i, pl.ds(j * 16, 16)] = jnp.zeros((16,), jnp.float32)
        return carry

    lax.fori_loop(0, CHUNK, _zrow, 0)

    def _zcopy(k, carry):
        pltpu.sync_copy(
            msgv[0], agg_sh.at[pl.ds(s * ROWS_PER_TILE + k * CHUNK, CHUNK)]
        )
        return carry

    lax.fori_loop(0, ROWS_PER_TILE // CHUNK, _zcopy, 0)

    wait_src(0)
    issue_data(0, 0)
    wait_src(1)
    issue_data(1, 1)
    plsc.subcore_barrier()

    wait_data(0, 0)
    compute(0)
    wait_dst(0)
    issue_scatter(0)
    wait_src(2)
    issue_data(2, 2)
    issue_idx(3, 3)

    wait_data(1, 1)
    compute(1)
    wait_dst(1)
    issue_scatter(1)
    wait_src(3)
    issue_data(3, 3)
    issue_idx(4, 4)

    def _block(i, stat):
        wait_data(i, stat)
        wait_scatter(stat)
        compute(stat)
        wait_dst(stat)
        issue_scatter(stat)

        @pl.when(i + 2 < N_CHUNKS)
        def _():
            wait_src(stat + 2)
            issue_data(i + 2, stat + 2)

        @pl.when(i + 3 < N_CHUNKS)
        def _():
            issue_idx(i + 3, stat + 3)

    def _oct(g, carry):
        i0 = 2 + 8 * g
        for k in range(8):
            _block(i0 + k, 2 + k)
        return carry

    lax.fori_loop(0, (N_CHUNKS - 2) // 8, _oct, 0)

    wait_scatter(0)
    wait_scatter(1)
    plsc.subcore_barrier()

    pltpu.sync_copy(
        agg_sh.at[pl.ds(s * ROWS_PER_TILE, ROWS_PER_TILE)],
        out_hbm.at[c, pl.ds(s * ROWS_PER_TILE, ROWS_PER_TILE)],
    )


_INV_SQRT_N = 1.0 / math.sqrt(float(N_NODES))


def _out_body(agg_ref, x_ref, w_ref, b_ref, g_ref, bt_ref, o_ref):
    agg = agg_ref[0] + agg_ref[1]
    out = jnp.dot(agg, w_ref[...], preferred_element_type=jnp.float32) + b_ref[...]
    mean = jnp.mean(out, axis=-1, keepdims=True)
    var = jnp.mean((out - mean) ** 2, axis=-1, keepdims=True)
    out = (out - mean) * lax.rsqrt(var + 1e-5) * g_ref[...] + bt_ref[...]
    o_ref[...] = out * _INV_SQRT_N + x_ref[...]


def _finish(agg_parts, node_feat, W_lin, b_lin, gamma, beta):
    BN = 1000
    return pl.pallas_call(
        _out_body,
        grid=(N_NODES // BN,),
        in_specs=[
            pl.BlockSpec((NC, BN, D), lambda i: (0, i, 0)),
            pl.BlockSpec((BN, D), lambda i: (i, 0)),
            pl.BlockSpec((D, D), lambda i: (0, 0)),
            pl.BlockSpec((1, D), lambda i: (0, 0)),
            pl.BlockSpec((1, D), lambda i: (0, 0)),
            pl.BlockSpec((1, D), lambda i: (0, 0)),
        ],
        out_specs=pl.BlockSpec((BN, D), lambda i: (i, 0)),
        out_shape=jax.ShapeDtypeStruct((N_NODES, D), jnp.float32),
    )(agg_parts, node_feat, W_lin, b_lin.reshape(1, D),
      gamma.reshape(1, D), beta.reshape(1, D))


def kernel(node_feat, edge_attr, W_edge, b_edge, W_lin, b_lin, gamma, beta,
           edge_index):
    src = edge_index[0].astype(jnp.int32)
    dst = edge_index[1].astype(jnp.int32)
    emb = _edge_embed(edge_attr, W_edge, b_edge)
    agg_parts = _sc_aggregate(node_feat, emb, src, dst)
    return _finish(agg_parts, node_feat, W_lin, b_lin, gamma, beta)
